# baseline (speedup 1.0000x reference)
import jax
import jax.numpy as jnp
from jax.experimental import pallas as pl

_T_TILE = 512


def _add_pe_kernel(x_ref, pe_ref, o_ref):
    o_ref[0] = x_ref[0] + pe_ref[...]


def kernel(x, pe):
    b, t, d = x.shape
    t_tiles = t // _T_TILE
    grid = (t_tiles, b)
    return pl.pallas_call(
        _add_pe_kernel,
        grid=grid,
        in_specs=[
            pl.BlockSpec((1, _T_TILE, d), lambda tt, bb: (bb, tt, 0)),
            pl.BlockSpec((_T_TILE, d), lambda tt, bb: (tt, 0)),
        ],
        out_specs=pl.BlockSpec((1, _T_TILE, d), lambda tt, bb: (bb, tt, 0)),
        out_shape=jax.ShapeDtypeStruct((b, t, d), x.dtype),
    )(x, pe)


# speedup vs baseline: 1.1701x; 1.1701x over previous
import jax
import jax.numpy as jnp
from jax.experimental import pallas as pl

_T_TILE = 2048


def _add_pe_kernel(x_ref, pe_ref, o_ref):
    o_ref[0] = x_ref[0] + pe_ref[...]


def kernel(x, pe):
    b, t, d = x.shape
    t_tiles = t // _T_TILE
    grid = (t_tiles, b)
    return pl.pallas_call(
        _add_pe_kernel,
        grid=grid,
        in_specs=[
            pl.BlockSpec((1, _T_TILE, d), lambda tt, bb: (bb, tt, 0)),
            pl.BlockSpec((_T_TILE, d), lambda tt, bb: (tt, 0)),
        ],
        out_specs=pl.BlockSpec((1, _T_TILE, d), lambda tt, bb: (bb, tt, 0)),
        out_shape=jax.ShapeDtypeStruct((b, t, d), x.dtype),
    )(x, pe)
